# initial kernel scaffold (unmeasured)
import jax
import jax.numpy as jnp
from jax import lax
from jax.experimental import pallas as pl
from jax.experimental.pallas import tpu as pltpu


def kernel(
    x,
):
    def body(*refs):
        pass

    out_shape = jax.ShapeDtypeStruct(..., jnp.float32)
    return pl.pallas_call(body, out_shape=out_shape)(...)


# baseline (device time: 43471 ns/iter reference)
import jax
import jax.numpy as jnp
from jax import lax
from jax.experimental import pallas as pl
from jax.experimental.pallas import tpu as pltpu

N_DEV = 32


def kernel(x):
    m_per, n = x.shape
    chunk = m_per // N_DEV

    def body(
        x_ref,
        out_ref,
        xbf_ref,
        rs_recv_ref,
        ag_src_ref,
        ag_recv_ref,
        rs_send_sems,
        rs_recv_sems,
        ag_send_sems,
        ag_recv_sems,
    ):
        d = lax.axis_index("i")

        xbf_ref[...] = (
            x_ref[...].astype(jnp.bfloat16).reshape(N_DEV, chunk, n)
        )

        for off in range(1, N_DEV):
            tgt = lax.rem(d + off, N_DEV)
            pltpu.make_async_remote_copy(
                src_ref=xbf_ref.at[tgt],
                dst_ref=rs_recv_ref.at[d],
                send_sem=rs_send_sems.at[tgt],
                recv_sem=rs_recv_sems.at[d],
                device_id=(tgt,),
                device_id_type=pl.DeviceIdType.MESH,
            ).start()
        rs_recv_ref[d] = xbf_ref[d]

        for off in range(1, N_DEV):
            src_dev = lax.rem(d + off, N_DEV)
            pltpu.make_async_remote_copy(
                src_ref=rs_recv_ref.at[src_dev],
                dst_ref=rs_recv_ref.at[src_dev],
                send_sem=rs_send_sems.at[src_dev],
                recv_sem=rs_recv_sems.at[src_dev],
                device_id=(d,),
                device_id_type=pl.DeviceIdType.MESH,
            ).wait_recv()

        acc = jnp.sum(rs_recv_ref[...].astype(jnp.float32), axis=0)
        ag_src_ref[...] = acc.astype(jnp.bfloat16)

        for off in range(1, N_DEV):
            tgt = lax.rem(d + off, N_DEV)
            pltpu.make_async_remote_copy(
                src_ref=ag_src_ref,
                dst_ref=ag_recv_ref.at[d],
                send_sem=ag_send_sems.at[tgt],
                recv_sem=ag_recv_sems.at[d],
                device_id=(tgt,),
                device_id_type=pl.DeviceIdType.MESH,
            ).start()
        ag_recv_ref[d] = ag_src_ref[...]

        for off in range(1, N_DEV):
            src_dev = lax.rem(d + off, N_DEV)
            pltpu.make_async_remote_copy(
                src_ref=ag_recv_ref.at[src_dev],
                dst_ref=ag_recv_ref.at[src_dev],
                send_sem=ag_send_sems.at[src_dev],
                recv_sem=ag_recv_sems.at[src_dev],
                device_id=(d,),
                device_id_type=pl.DeviceIdType.MESH,
            ).wait_recv()

        out_ref[...] = (
            ag_recv_ref[...].astype(jnp.float32).reshape(m_per, n)
        )

        for off in range(1, N_DEV):
            tgt = lax.rem(d + off, N_DEV)
            pltpu.make_async_remote_copy(
                src_ref=xbf_ref.at[tgt],
                dst_ref=rs_recv_ref.at[d],
                send_sem=rs_send_sems.at[tgt],
                recv_sem=rs_recv_sems.at[d],
                device_id=(tgt,),
                device_id_type=pl.DeviceIdType.MESH,
            ).wait_send()
            pltpu.make_async_remote_copy(
                src_ref=ag_src_ref,
                dst_ref=ag_recv_ref.at[d],
                send_sem=ag_send_sems.at[tgt],
                recv_sem=ag_recv_sems.at[d],
                device_id=(tgt,),
                device_id_type=pl.DeviceIdType.MESH,
            ).wait_send()

    return pl.pallas_call(
        body,
        out_shape=jax.ShapeDtypeStruct((m_per, n), jnp.float32),
        in_specs=[pl.BlockSpec(memory_space=pltpu.VMEM)],
        out_specs=pl.BlockSpec(memory_space=pltpu.VMEM),
        scratch_shapes=[
            pltpu.VMEM((N_DEV, chunk, n), jnp.bfloat16),
            pltpu.VMEM((N_DEV, chunk, n), jnp.bfloat16),
            pltpu.VMEM((chunk, n), jnp.bfloat16),
            pltpu.VMEM((N_DEV, chunk, n), jnp.bfloat16),
            pltpu.SemaphoreType.DMA((N_DEV,)),
            pltpu.SemaphoreType.DMA((N_DEV,)),
            pltpu.SemaphoreType.DMA((N_DEV,)),
            pltpu.SemaphoreType.DMA((N_DEV,)),
        ],
    )(x)


# device time: 37405 ns/iter; 1.1622x vs baseline; 1.1622x over previous
import jax
import jax.numpy as jnp
from jax import lax
from jax.experimental import pallas as pl
from jax.experimental.pallas import tpu as pltpu

N_DEV = 32
NSEG = 2


def kernel(x):
    m_per, n = x.shape
    chunk = m_per // N_DEV
    seg = chunk // NSEG

    def body(
        x_ref,
        out_ref,
        xbf_ref,
        rs_recv_ref,
        acc_ref,
        ag_recv_ref,
        rs_send_sems,
        rs_recv_sems,
        ag_send_sems,
        ag_recv_sems,
    ):
        d = lax.axis_index("i")

        barrier_sem = pltpu.get_barrier_semaphore()
        for off in range(1, N_DEV):
            tgt = lax.rem(d + off, N_DEV)
            pl.semaphore_signal(
                barrier_sem,
                inc=1,
                device_id=(tgt,),
                device_id_type=pl.DeviceIdType.MESH,
            )

        xbf_ref[...] = (
            x_ref[...].astype(jnp.bfloat16).reshape(N_DEV, NSEG, seg, n)
        )

        pl.semaphore_wait(barrier_sem, N_DEV - 1)

        for s in range(NSEG):
            for off in range(1, N_DEV):
                tgt = lax.rem(d + off, N_DEV)
                pltpu.make_async_remote_copy(
                    src_ref=xbf_ref.at[tgt, s],
                    dst_ref=rs_recv_ref.at[d, s],
                    send_sem=rs_send_sems.at[tgt, s],
                    recv_sem=rs_recv_sems.at[d, s],
                    device_id=(tgt,),
                    device_id_type=pl.DeviceIdType.MESH,
                ).start()
        for s in range(NSEG):
            rs_recv_ref[d, s] = xbf_ref[d, s]

        for s in range(NSEG):
            for off in range(1, N_DEV):
                src_dev = lax.rem(d + off, N_DEV)
                pltpu.make_async_remote_copy(
                    src_ref=rs_recv_ref.at[src_dev, s],
                    dst_ref=rs_recv_ref.at[src_dev, s],
                    send_sem=rs_send_sems.at[src_dev, s],
                    recv_sem=rs_recv_sems.at[src_dev, s],
                    device_id=(d,),
                    device_id_type=pl.DeviceIdType.MESH,
                ).wait_recv()
            acc = jnp.sum(rs_recv_ref[:, s].astype(jnp.float32), axis=0)
            acc_ref[s] = acc.astype(jnp.bfloat16)
            for off in range(1, N_DEV):
                tgt = lax.rem(d + off, N_DEV)
                pltpu.make_async_remote_copy(
                    src_ref=acc_ref.at[s],
                    dst_ref=ag_recv_ref.at[d, s],
                    send_sem=ag_send_sems.at[tgt, s],
                    recv_sem=ag_recv_sems.at[d, s],
                    device_id=(tgt,),
                    device_id_type=pl.DeviceIdType.MESH,
                ).start()
            ag_recv_ref[d, s] = acc_ref[s]

        for s in range(NSEG):
            for off in range(1, N_DEV):
                src_dev = lax.rem(d + off, N_DEV)
                pltpu.make_async_remote_copy(
                    src_ref=ag_recv_ref.at[src_dev, s],
                    dst_ref=ag_recv_ref.at[src_dev, s],
                    send_sem=ag_send_sems.at[src_dev, s],
                    recv_sem=ag_recv_sems.at[src_dev, s],
                    device_id=(d,),
                    device_id_type=pl.DeviceIdType.MESH,
                ).wait_recv()

        out_ref[...] = (
            ag_recv_ref[...].astype(jnp.float32).reshape(m_per, n)
        )

        for s in range(NSEG):
            for off in range(1, N_DEV):
                tgt = lax.rem(d + off, N_DEV)
                pltpu.make_async_remote_copy(
                    src_ref=xbf_ref.at[tgt, s],
                    dst_ref=rs_recv_ref.at[d, s],
                    send_sem=rs_send_sems.at[tgt, s],
                    recv_sem=rs_recv_sems.at[d, s],
                    device_id=(tgt,),
                    device_id_type=pl.DeviceIdType.MESH,
                ).wait_send()
                pltpu.make_async_remote_copy(
                    src_ref=acc_ref.at[s],
                    dst_ref=ag_recv_ref.at[d, s],
                    send_sem=ag_send_sems.at[tgt, s],
                    recv_sem=ag_recv_sems.at[d, s],
                    device_id=(tgt,),
                    device_id_type=pl.DeviceIdType.MESH,
                ).wait_send()


    return pl.pallas_call(
        body,
        out_shape=jax.ShapeDtypeStruct((m_per, n), jnp.float32),
        in_specs=[pl.BlockSpec(memory_space=pltpu.VMEM)],
        out_specs=pl.BlockSpec(memory_space=pltpu.VMEM),
        scratch_shapes=[
            pltpu.VMEM((N_DEV, NSEG, seg, n), jnp.bfloat16),
            pltpu.VMEM((N_DEV, NSEG, seg, n), jnp.bfloat16),
            pltpu.VMEM((NSEG, seg, n), jnp.bfloat16),
            pltpu.VMEM((N_DEV, NSEG, seg, n), jnp.bfloat16),
            pltpu.SemaphoreType.DMA((N_DEV, NSEG)),
            pltpu.SemaphoreType.DMA((N_DEV, NSEG)),
            pltpu.SemaphoreType.DMA((N_DEV, NSEG)),
            pltpu.SemaphoreType.DMA((N_DEV, NSEG)),
        ],
        compiler_params=pltpu.CompilerParams(collective_id=0),
    )(x)
